# baseline (speedup 1.0000x reference)
import jax
import jax.numpy as jnp
from jax.experimental import pallas as pl
from jax.experimental.pallas import tpu as pltpu

EPS = 0.001
R = 1024


def _body(ids_ref, mean_ref, std_ref, x_hbm, o_hbm,
          buf, scale_s, mean_s, in_sems, out_sems):
    B = 4
    S_PER_B = 2048
    nchunks = (B * S_PER_B) // R
    chunks_per_b = S_PER_B // R

    for c in range(nchunks):
        pltpu.make_async_copy(
            x_hbm.at[pl.ds(c * R, R), :], buf.at[c], in_sems.at[c]
        ).start()

    for b in range(B):
        idx = ids_ref[b]
        mean_s[pl.ds(b, 1), :] = mean_ref[pl.ds(idx, 1), :]
        scale_s[pl.ds(b, 1), :] = 1.0 / (jnp.exp(std_ref[pl.ds(idx, 1), :]) + EPS)

    for c in range(nchunks):
        b = c // chunks_per_b
        pltpu.make_async_copy(
            x_hbm.at[pl.ds(c * R, R), :], buf.at[c], in_sems.at[c]
        ).wait()
        buf[c] = (buf[c] - mean_s[pl.ds(b, 1), :]) * scale_s[pl.ds(b, 1), :]
        pltpu.make_async_copy(
            buf.at[c], o_hbm.at[pl.ds(c * R, R), :], out_sems.at[c]
        ).start()

    for c in range(nchunks):
        pltpu.make_async_copy(
            buf.at[c], o_hbm.at[pl.ds(c * R, R), :], out_sems.at[c]
        ).wait()


def kernel(x, context_id, initial_mean, initial_std):
    B, S, D = x.shape
    nchunks = (B * S) // R
    ids = context_id.reshape(-1)
    x2 = x.reshape(B * S, D)
    out = pl.pallas_call(
        _body,
        grid=(),
        in_specs=[
            pl.BlockSpec(memory_space=pltpu.SMEM),
            pl.BlockSpec(memory_space=pltpu.VMEM),
            pl.BlockSpec(memory_space=pltpu.VMEM),
            pl.BlockSpec(memory_space=pl.ANY),
        ],
        out_specs=pl.BlockSpec(memory_space=pl.ANY),
        out_shape=jax.ShapeDtypeStruct((B * S, D), x.dtype),
        scratch_shapes=[
            pltpu.VMEM((nchunks, R, D), jnp.float32),
            pltpu.VMEM((B, D), jnp.float32),
            pltpu.VMEM((B, D), jnp.float32),
            pltpu.SemaphoreType.DMA((nchunks,)),
            pltpu.SemaphoreType.DMA((nchunks,)),
        ],
    )(ids, initial_mean, initial_std, x2)
    return out.reshape(B, S, D)


# speedup vs baseline: 1.0123x; 1.0123x over previous
import jax
import jax.numpy as jnp
from jax.experimental import pallas as pl
from jax.experimental.pallas import tpu as pltpu

EPS = 0.001
R = 2048


def _body(ids_ref, mean_ref, std_ref, x_hbm, o_hbm,
          buf, scale_s, mean_s, in_sems, out_sems):
    B = 4
    S_PER_B = 2048
    nchunks = (B * S_PER_B) // R
    chunks_per_b = S_PER_B // R

    for c in range(nchunks):
        pltpu.make_async_copy(
            x_hbm.at[pl.ds(c * R, R), :], buf.at[c], in_sems.at[c]
        ).start()

    for b in range(B):
        idx = ids_ref[b]
        mean_s[pl.ds(b, 1), :] = mean_ref[pl.ds(idx, 1), :]
        scale_s[pl.ds(b, 1), :] = 1.0 / (jnp.exp(std_ref[pl.ds(idx, 1), :]) + EPS)

    for c in range(nchunks):
        b = c // chunks_per_b
        pltpu.make_async_copy(
            x_hbm.at[pl.ds(c * R, R), :], buf.at[c], in_sems.at[c]
        ).wait()
        buf[c] = (buf[c] - mean_s[pl.ds(b, 1), :]) * scale_s[pl.ds(b, 1), :]
        pltpu.make_async_copy(
            buf.at[c], o_hbm.at[pl.ds(c * R, R), :], out_sems.at[c]
        ).start()

    for c in range(nchunks):
        pltpu.make_async_copy(
            buf.at[c], o_hbm.at[pl.ds(c * R, R), :], out_sems.at[c]
        ).wait()


def kernel(x, context_id, initial_mean, initial_std):
    B, S, D = x.shape
    nchunks = (B * S) // R
    ids = context_id.reshape(-1)
    x2 = x.reshape(B * S, D)
    out = pl.pallas_call(
        _body,
        grid=(),
        in_specs=[
            pl.BlockSpec(memory_space=pltpu.SMEM),
            pl.BlockSpec(memory_space=pltpu.VMEM),
            pl.BlockSpec(memory_space=pltpu.VMEM),
            pl.BlockSpec(memory_space=pl.ANY),
        ],
        out_specs=pl.BlockSpec(memory_space=pl.ANY),
        out_shape=jax.ShapeDtypeStruct((B * S, D), x.dtype),
        scratch_shapes=[
            pltpu.VMEM((nchunks, R, D), jnp.float32),
            pltpu.VMEM((B, D), jnp.float32),
            pltpu.VMEM((B, D), jnp.float32),
            pltpu.SemaphoreType.DMA((nchunks,)),
            pltpu.SemaphoreType.DMA((nchunks,)),
        ],
    )(ids, initial_mean, initial_std, x2)
    return out.reshape(B, S, D)


# speedup vs baseline: 1.0238x; 1.0114x over previous
import jax
import jax.numpy as jnp
from jax import lax
from jax.experimental import pallas as pl
from jax.experimental.pallas import tpu as pltpu

EPS = 0.001
R = 2048
DEPTH = 2


def _body(ids_ref, mean_ref, std_ref, x_hbm, o_hbm,
          in_buf, out_buf, scale_s, mean_s, in_sems, out_sems):
    B = 4
    S_PER_B = 2048
    nchunks = (B * S_PER_B) // R
    chunks_per_b = S_PER_B // R

    for b in range(B):
        idx = ids_ref[b]
        m = mean_ref[pl.ds(idx, 1), :]
        s = std_ref[pl.ds(idx, 1), :]
        mean_s[pl.ds(b, 1), :] = m
        scale_s[pl.ds(b, 1), :] = 1.0 / (jnp.exp(s) + EPS)

    for s in range(DEPTH):
        pltpu.make_async_copy(
            x_hbm.at[pl.ds(s * R, R), :], in_buf.at[s], in_sems.at[s]
        ).start()

    def outer(o, _):
        for s in range(DEPTH):
            c = o * DEPTH + s
            b = c // chunks_per_b
            pltpu.make_async_copy(
                x_hbm.at[pl.ds(c * R, R), :], in_buf.at[s], in_sems.at[s]
            ).wait()

            @pl.when(c >= DEPTH)
            def _():
                pltpu.make_async_copy(
                    out_buf.at[s], o_hbm.at[pl.ds((c - DEPTH) * R, R), :],
                    out_sems.at[s]
                ).wait()

            mrow = mean_s[pl.ds(b, 1), :]
            srow = scale_s[pl.ds(b, 1), :]
            out_buf[s] = (in_buf[s] - mrow) * srow

            pltpu.make_async_copy(
                out_buf.at[s], o_hbm.at[pl.ds(c * R, R), :], out_sems.at[s]
            ).start()

            @pl.when(c + DEPTH < nchunks)
            def _():
                pltpu.make_async_copy(
                    x_hbm.at[pl.ds((c + DEPTH) * R, R), :], in_buf.at[s],
                    in_sems.at[s]
                ).start()
        return ()

    lax.fori_loop(0, nchunks // DEPTH, outer, (), unroll=False)

    for s in range(DEPTH):
        c = nchunks - DEPTH + s
        pltpu.make_async_copy(
            out_buf.at[s], o_hbm.at[pl.ds(c * R, R), :], out_sems.at[s]
        ).wait()


def kernel(x, context_id, initial_mean, initial_std):
    B, S, D = x.shape
    ids = context_id.reshape(-1)
    x2 = x.reshape(B * S, D)
    out = pl.pallas_call(
        _body,
        grid=(),
        in_specs=[
            pl.BlockSpec(memory_space=pltpu.SMEM),
            pl.BlockSpec(memory_space=pltpu.VMEM),
            pl.BlockSpec(memory_space=pltpu.VMEM),
            pl.BlockSpec(memory_space=pl.ANY),
        ],
        out_specs=pl.BlockSpec(memory_space=pl.ANY),
        out_shape=jax.ShapeDtypeStruct((B * S, D), x.dtype),
        scratch_shapes=[
            pltpu.VMEM((DEPTH, R, D), jnp.float32),
            pltpu.VMEM((DEPTH, R, D), jnp.float32),
            pltpu.VMEM((B, D), jnp.float32),
            pltpu.VMEM((B, D), jnp.float32),
            pltpu.SemaphoreType.DMA((DEPTH,)),
            pltpu.SemaphoreType.DMA((DEPTH,)),
        ],
    )(ids, initial_mean, initial_std, x2)
    return out.reshape(B, S, D)
